# baseline (speedup 1.0000x reference)
import functools

import jax
import jax.numpy as jnp
from jax import lax
from jax.experimental import pallas as pl
from jax.experimental.pallas import tpu as pltpu
from jax.experimental.pallas import tpu_sc as plsc

N_POINTS = 1600000
NUM_SEG = 1024
ACC = NUM_SEG * 3

_info = plsc.get_sparse_core_info()
NUM_CORES = _info.num_cores
NUM_SUBCORES = _info.num_subcores
NW = NUM_CORES * NUM_SUBCORES
ROWS_PER_W = N_POINTS // NW
CHUNK = 10000
NCHUNK = ROWS_PER_W // CHUNK
VREGS = CHUNK // 16
U = 5
BLK = U * 16
NBLK = VREGS // U


def _tree_sum(vs):
  while len(vs) > 1:
    vs = [a + b for a, b in zip(vs[::2], vs[1::2])] + (
        [vs[-1]] if len(vs) % 2 else [])
  return vs[0]


def _sc_partials(pt, q, batch):
  mesh = plsc.VectorSubcoreMesh(core_axis_name="c", subcore_axis_name="s")

  @functools.partial(
      pl.kernel,
      mesh=mesh,
      compiler_params=pltpu.CompilerParams(needs_layout_passes=False),
      out_type=[
          jax.ShapeDtypeStruct((NW, ACC), jnp.float32),
          jax.ShapeDtypeStruct((NW, ACC), jnp.float32),
          jax.ShapeDtypeStruct((NW, 16), jnp.float32),
      ],
      scratch_types=[
          pltpu.VMEM((CHUNK,), jnp.float32),
          pltpu.VMEM((CHUNK,), jnp.float32),
          pltpu.VMEM((CHUNK,), jnp.float32),
          pltpu.VMEM((CHUNK,), jnp.float32),
          pltpu.VMEM((CHUNK,), jnp.int32),
          pltpu.VMEM((ACC,), jnp.float32),
          pltpu.VMEM((ACC,), jnp.float32),
          pltpu.VMEM((16,), jnp.float32),
      ],
  )
  def body(pt_hbm, q_hbm, b_hbm, out_qp, out_p, out_qs,
           xbuf, ybuf, zbuf, qbuf, bbuf, acc_qp, acc_p, qs_buf):
    wid = lax.axis_index("s") * NUM_CORES + lax.axis_index("c")
    row0 = wid * ROWS_PER_W

    zeros = jnp.zeros((16,), jnp.float32)
    zeros_i = jnp.zeros((16,), jnp.int32)

    def zero_body(j, _):
      acc_qp[pl.ds(j * 16, 16)] = zeros
      acc_p[pl.ds(j * 16, 16)] = zeros
      return 0

    lax.fori_loop(0, ACC // 16, zero_body, 0)

    def flush(cur, vqx, vqy, vqz, vpx, vpy, vpz):
      i0 = zeros_i + jnp.maximum(cur, 0) * 3
      plsc.addupdate_scatter(acc_qp, [i0], vqx)
      plsc.addupdate_scatter(acc_qp, [i0 + 1], vqy)
      plsc.addupdate_scatter(acc_qp, [i0 + 2], vqz)
      plsc.addupdate_scatter(acc_p, [i0], vpx)
      plsc.addupdate_scatter(acc_p, [i0 + 1], vpy)
      plsc.addupdate_scatter(acc_p, [i0 + 2], vpz)

    def chunk_body(c, carry):
      r0 = row0 + c * CHUNK
      pltpu.sync_copy(pt_hbm.at[pl.ds(r0, CHUNK)], xbuf)
      pltpu.sync_copy(pt_hbm.at[pl.ds(N_POINTS + r0, CHUNK)], ybuf)
      pltpu.sync_copy(pt_hbm.at[pl.ds(2 * N_POINTS + r0, CHUNK)], zbuf)
      pltpu.sync_copy(q_hbm.at[pl.ds(r0, CHUNK)], qbuf)
      pltpu.sync_copy(b_hbm.at[pl.ds(r0, CHUNK)], bbuf)

      def blk_body(k, carry):
        cur, vqx, vqy, vqz, vpx, vpy, vpz, vqs = carry
        k0 = k * BLK
        b_first = bbuf[pl.ds(k0, 16)][0]
        b_last = bbuf[pl.ds(k0 + BLK - 16, 16)][15]
        is_fast = jnp.logical_and(b_first == cur, b_last == b_first)

        def fast_fn(carry):
          cur, vqx, vqy, vqz, vpx, vpy, vpz, vqs = carry
          xs, ys, zs, qs, qxs, qys, qzs = [], [], [], [], [], [], []
          for u in range(U):
            o = k0 + u * 16
            xv = xbuf[pl.ds(o, 16)]
            yv = ybuf[pl.ds(o, 16)]
            zv = zbuf[pl.ds(o, 16)]
            qv = qbuf[pl.ds(o, 16)]
            xs.append(xv); ys.append(yv); zs.append(zv); qs.append(qv)
            qxs.append(qv * xv); qys.append(qv * yv); qzs.append(qv * zv)
          return (cur,
                  vqx + _tree_sum(qxs), vqy + _tree_sum(qys),
                  vqz + _tree_sum(qzs),
                  vpx + _tree_sum(xs), vpy + _tree_sum(ys),
                  vpz + _tree_sum(zs), vqs + _tree_sum(qs))

        def slow_fn(carry):
          cur, vqx, vqy, vqz, vpx, vpy, vpz, vqs = carry
          flush(cur, vqx, vqy, vqz, vpx, vpy, vpz)
          qs = []
          for u in range(U):
            o = k0 + u * 16
            xv = xbuf[pl.ds(o, 16)]
            yv = ybuf[pl.ds(o, 16)]
            zv = zbuf[pl.ds(o, 16)]
            qv = qbuf[pl.ds(o, 16)]
            b3 = bbuf[pl.ds(o, 16)] * 3
            plsc.addupdate_scatter(acc_qp, [b3], qv * xv)
            plsc.addupdate_scatter(acc_qp, [b3 + 1], qv * yv)
            plsc.addupdate_scatter(acc_qp, [b3 + 2], qv * zv)
            plsc.addupdate_scatter(acc_p, [b3], xv)
            plsc.addupdate_scatter(acc_p, [b3 + 1], yv)
            plsc.addupdate_scatter(acc_p, [b3 + 2], zv)
            qs.append(qv)
          return (b_last, zeros, zeros, zeros, zeros, zeros, zeros,
                  vqs + _tree_sum(qs))

        return lax.cond(is_fast, fast_fn, slow_fn, carry)

      return lax.fori_loop(0, NBLK, blk_body, carry)

    carry0 = (jnp.int32(-1), zeros, zeros, zeros, zeros, zeros, zeros, zeros)
    cur, vqx, vqy, vqz, vpx, vpy, vpz, vqs = lax.fori_loop(
        0, NCHUNK, chunk_body, carry0)
    flush(cur, vqx, vqy, vqz, vpx, vpy, vpz)
    qs_buf[...] = vqs

    pltpu.sync_copy(acc_qp, out_qp.at[wid])
    pltpu.sync_copy(acc_p, out_p.at[wid])
    pltpu.sync_copy(qs_buf, out_qs.at[wid])

  return body(pt, q, batch)


def _combine_body(qp_ref, p_ref, qs_ref, out_ref):
  m = jnp.sum(qs_ref[...]) * (1.0 / N_POINTS)
  out_ref[...] = (jnp.sum(qp_ref[...], axis=0, keepdims=True)
                  - m * jnp.sum(p_ref[...], axis=0, keepdims=True))


def kernel(positions, q, batch):
  qp_part, p_part, qs_part = _sc_partials(positions.T.reshape(-1), q, batch)
  out = pl.pallas_call(
      _combine_body,
      out_shape=jax.ShapeDtypeStruct((1, ACC), jnp.float32),
  )(qp_part, p_part, qs_part)
  return out.reshape(NUM_SEG, 3)


# speedup vs baseline: 3.0393x; 3.0393x over previous
import functools

import jax
import jax.numpy as jnp
from jax import lax
from jax.experimental import pallas as pl
from jax.experimental.pallas import tpu as pltpu
from jax.experimental.pallas import tpu_sc as plsc

N_POINTS = 1600000
NUM_SEG = 1024
ACC = NUM_SEG * 3

_info = plsc.get_sparse_core_info()
NUM_CORES = _info.num_cores
NUM_SUBCORES = _info.num_subcores
NW = NUM_CORES * NUM_SUBCORES
ROWS_PER_W = N_POINTS // NW
CHUNK = 10000
NCHUNK = ROWS_PER_W // CHUNK
VREGS = CHUNK // 16
U = 5
BLK = U * 16
NBLK = VREGS // U


def _tree_sum(vs):
  while len(vs) > 1:
    vs = [a + b for a, b in zip(vs[::2], vs[1::2])] + (
        [vs[-1]] if len(vs) % 2 else [])
  return vs[0]


def _sc_partials(x, y, z, q, batch):
  mesh = plsc.VectorSubcoreMesh(core_axis_name="c", subcore_axis_name="s")

  @functools.partial(
      pl.kernel,
      mesh=mesh,
      compiler_params=pltpu.CompilerParams(needs_layout_passes=False),
      out_type=[
          jax.ShapeDtypeStruct((NW, ACC), jnp.float32),
          jax.ShapeDtypeStruct((NW, ACC), jnp.float32),
          jax.ShapeDtypeStruct((NW, 16), jnp.float32),
      ],
      scratch_types=[
          pltpu.VMEM((CHUNK,), jnp.float32),
          pltpu.VMEM((CHUNK,), jnp.float32),
          pltpu.VMEM((CHUNK,), jnp.float32),
          pltpu.VMEM((CHUNK,), jnp.float32),
          pltpu.VMEM((CHUNK,), jnp.int32),
          pltpu.VMEM((CHUNK,), jnp.float32),
          pltpu.VMEM((CHUNK,), jnp.float32),
          pltpu.VMEM((CHUNK,), jnp.float32),
          pltpu.VMEM((CHUNK,), jnp.float32),
          pltpu.VMEM((CHUNK,), jnp.int32),
          pltpu.VMEM((ACC,), jnp.float32),
          pltpu.VMEM((ACC,), jnp.float32),
          pltpu.VMEM((16,), jnp.float32),
          pltpu.SemaphoreType.DMA,
          pltpu.SemaphoreType.DMA,
      ],
  )
  def body(x_hbm, y_hbm, z_hbm, q_hbm, b_hbm, out_qp, out_p, out_qs,
           xbuf0, ybuf0, zbuf0, qbuf0, bbuf0,
           xbuf1, ybuf1, zbuf1, qbuf1, bbuf1,
           acc_qp, acc_p, qs_buf, sem0, sem1):
    wid = lax.axis_index("s") * NUM_CORES + lax.axis_index("c")
    row0 = wid * ROWS_PER_W

    zeros = jnp.zeros((16,), jnp.float32)
    zeros_i = jnp.zeros((16,), jnp.int32)

    def zero_body(j, _):
      acc_qp[pl.ds(j * 16, 16)] = zeros
      acc_p[pl.ds(j * 16, 16)] = zeros
      return 0

    lax.fori_loop(0, ACC // 16, zero_body, 0)

    def flush(cur, vqx, vqy, vqz, vpx, vpy, vpz):
      i0 = zeros_i + jnp.maximum(cur, 0) * 3
      plsc.addupdate_scatter(acc_qp, [i0], vqx)
      plsc.addupdate_scatter(acc_qp, [i0 + 1], vqy)
      plsc.addupdate_scatter(acc_qp, [i0 + 2], vqz)
      plsc.addupdate_scatter(acc_p, [i0], vpx)
      plsc.addupdate_scatter(acc_p, [i0 + 1], vpy)
      plsc.addupdate_scatter(acc_p, [i0 + 2], vpz)

    bufsets = [(xbuf0, ybuf0, zbuf0, qbuf0, bbuf0, sem0),
               (xbuf1, ybuf1, zbuf1, qbuf1, bbuf1, sem1)]

    def start_dmas(c, bufset):
      r0 = row0 + c * CHUNK
      xb, yb, zb, qb, bb, sem = bufset
      return [
          pltpu.async_copy(x_hbm.at[pl.ds(r0, CHUNK)], xb, sem),
          pltpu.async_copy(y_hbm.at[pl.ds(r0, CHUNK)], yb, sem),
          pltpu.async_copy(z_hbm.at[pl.ds(r0, CHUNK)], zb, sem),
          pltpu.async_copy(q_hbm.at[pl.ds(r0, CHUNK)], qb, sem),
          pltpu.async_copy(b_hbm.at[pl.ds(r0, CHUNK)], bb, sem),
      ]

    def make_blk_body(bufset):
      xbuf, ybuf, zbuf, qbuf, bbuf, _ = bufset

      def blk_body(k, carry):
        cur, vqx, vqy, vqz, vpx, vpy, vpz, vqs = carry
        k0 = k * BLK
        b_first = bbuf[pl.ds(k0, 16)][0]
        b_last = bbuf[pl.ds(k0 + BLK - 16, 16)][15]
        is_fast = jnp.logical_and(b_first == cur, b_last == b_first)

        def fast_fn(carry):
          cur, vqx, vqy, vqz, vpx, vpy, vpz, vqs = carry
          xs, ys, zs, qs, qxs, qys, qzs = [], [], [], [], [], [], []
          for u in range(U):
            o = k0 + u * 16
            xv = xbuf[pl.ds(o, 16)]
            yv = ybuf[pl.ds(o, 16)]
            zv = zbuf[pl.ds(o, 16)]
            qv = qbuf[pl.ds(o, 16)]
            xs.append(xv); ys.append(yv); zs.append(zv); qs.append(qv)
            qxs.append(qv * xv); qys.append(qv * yv); qzs.append(qv * zv)
          return (cur,
                  vqx + _tree_sum(qxs), vqy + _tree_sum(qys),
                  vqz + _tree_sum(qzs),
                  vpx + _tree_sum(xs), vpy + _tree_sum(ys),
                  vpz + _tree_sum(zs), vqs + _tree_sum(qs))

        def slow_fn(carry):
          cur, vqx, vqy, vqz, vpx, vpy, vpz, vqs = carry
          flush(cur, vqx, vqy, vqz, vpx, vpy, vpz)
          qs = []
          for u in range(U):
            o = k0 + u * 16
            xv = xbuf[pl.ds(o, 16)]
            yv = ybuf[pl.ds(o, 16)]
            zv = zbuf[pl.ds(o, 16)]
            qv = qbuf[pl.ds(o, 16)]
            b3 = bbuf[pl.ds(o, 16)] * 3
            plsc.addupdate_scatter(acc_qp, [b3], qv * xv)
            plsc.addupdate_scatter(acc_qp, [b3 + 1], qv * yv)
            plsc.addupdate_scatter(acc_qp, [b3 + 2], qv * zv)
            plsc.addupdate_scatter(acc_p, [b3], xv)
            plsc.addupdate_scatter(acc_p, [b3 + 1], yv)
            plsc.addupdate_scatter(acc_p, [b3 + 2], zv)
            qs.append(qv)
          return (b_last, zeros, zeros, zeros, zeros, zeros, zeros,
                  vqs + _tree_sum(qs))

        return lax.cond(is_fast, fast_fn, slow_fn, carry)

      return blk_body

    carry = (jnp.int32(-1), zeros, zeros, zeros, zeros, zeros, zeros, zeros)
    handles = start_dmas(0, bufsets[0])
    for c in range(NCHUNK):
      for h in handles:
        h.wait()
      if c + 1 < NCHUNK:
        handles = start_dmas(c + 1, bufsets[(c + 1) % 2])
      carry = lax.fori_loop(0, NBLK, make_blk_body(bufsets[c % 2]), carry)
    cur, vqx, vqy, vqz, vpx, vpy, vpz, vqs = carry
    flush(cur, vqx, vqy, vqz, vpx, vpy, vpz)
    qs_buf[...] = vqs

    pltpu.sync_copy(acc_qp, out_qp.at[wid])
    pltpu.sync_copy(acc_p, out_p.at[wid])
    pltpu.sync_copy(qs_buf, out_qs.at[wid])

  return body(x, y, z, q, batch)


def _combine_body(qp_ref, p_ref, qs_ref, out_ref):
  m = jnp.sum(qs_ref[...]) * (1.0 / N_POINTS)
  out_ref[...] = (jnp.sum(qp_ref[...], axis=0, keepdims=True)
                  - m * jnp.sum(p_ref[...], axis=0, keepdims=True))


def kernel(positions, q, batch):
  x = positions[:, 0]
  y = positions[:, 1]
  z = positions[:, 2]
  qp_part, p_part, qs_part = _sc_partials(x, y, z, q, batch)
  out = pl.pallas_call(
      _combine_body,
      out_shape=jax.ShapeDtypeStruct((1, ACC), jnp.float32),
  )(qp_part, p_part, qs_part)
  return out.reshape(NUM_SEG, 3)


# speedup vs baseline: 3.1389x; 1.0328x over previous
import functools

import jax
import jax.numpy as jnp
from jax import lax
from jax.experimental import pallas as pl
from jax.experimental.pallas import tpu as pltpu
from jax.experimental.pallas import tpu_sc as plsc

N_POINTS = 1600000
NUM_SEG = 1024
ACC = NUM_SEG * 3

_info = plsc.get_sparse_core_info()
NUM_CORES = _info.num_cores
NUM_SUBCORES = _info.num_subcores
NW = NUM_CORES * NUM_SUBCORES
ROWS_PER_W = N_POINTS // NW
CHUNK = 10000
NCHUNK = ROWS_PER_W // CHUNK
VREGS = CHUNK // 16
U = 5
BLK = U * 16
NBLK = VREGS // U


def _tree_sum(vs):
  while len(vs) > 1:
    vs = [a + b for a, b in zip(vs[::2], vs[1::2])] + (
        [vs[-1]] if len(vs) % 2 else [])
  return vs[0]


def _sc_partials(x, y, z, q, batch):
  mesh = plsc.VectorSubcoreMesh(core_axis_name="c", subcore_axis_name="s")

  @functools.partial(
      pl.kernel,
      mesh=mesh,
      compiler_params=pltpu.CompilerParams(needs_layout_passes=False),
      out_type=[
          jax.ShapeDtypeStruct((NW, ACC), jnp.float32),
          jax.ShapeDtypeStruct((NW, ACC), jnp.float32),
          jax.ShapeDtypeStruct((NW, 16), jnp.float32),
      ],
      scratch_types=[
          pltpu.VMEM((CHUNK,), jnp.float32),
          pltpu.VMEM((CHUNK,), jnp.float32),
          pltpu.VMEM((CHUNK,), jnp.float32),
          pltpu.VMEM((CHUNK,), jnp.float32),
          pltpu.VMEM((CHUNK,), jnp.int32),
          pltpu.VMEM((CHUNK,), jnp.float32),
          pltpu.VMEM((CHUNK,), jnp.float32),
          pltpu.VMEM((CHUNK,), jnp.float32),
          pltpu.VMEM((CHUNK,), jnp.float32),
          pltpu.VMEM((CHUNK,), jnp.int32),
          pltpu.VMEM((ACC,), jnp.float32),
          pltpu.VMEM((ACC,), jnp.float32),
          pltpu.VMEM((16,), jnp.float32),
          pltpu.SemaphoreType.DMA,
          pltpu.SemaphoreType.DMA,
      ],
  )
  def body(x_hbm, y_hbm, z_hbm, q_hbm, b_hbm, out_qp, out_p, out_qs,
           xbuf0, ybuf0, zbuf0, qbuf0, bbuf0,
           xbuf1, ybuf1, zbuf1, qbuf1, bbuf1,
           acc_qp, acc_p, qs_buf, sem0, sem1):
    wid = lax.axis_index("s") * NUM_CORES + lax.axis_index("c")
    row0 = wid * ROWS_PER_W

    zeros = jnp.zeros((16,), jnp.float32)
    zeros_i = jnp.zeros((16,), jnp.int32)

    def zero_body(j, _):
      acc_qp[pl.ds(j * 16, 16)] = zeros
      acc_p[pl.ds(j * 16, 16)] = zeros
      return 0

    lax.fori_loop(0, ACC // 16, zero_body, 0)

    def flush(cur, vqx, vqy, vqz, vpx, vpy, vpz):
      i0 = zeros_i + jnp.maximum(cur, 0) * 3
      plsc.addupdate_scatter(acc_qp, [i0], vqx)
      plsc.addupdate_scatter(acc_qp, [i0 + 1], vqy)
      plsc.addupdate_scatter(acc_qp, [i0 + 2], vqz)
      plsc.addupdate_scatter(acc_p, [i0], vpx)
      plsc.addupdate_scatter(acc_p, [i0 + 1], vpy)
      plsc.addupdate_scatter(acc_p, [i0 + 2], vpz)

    bufsets = [(xbuf0, ybuf0, zbuf0, qbuf0, bbuf0, sem0),
               (xbuf1, ybuf1, zbuf1, qbuf1, bbuf1, sem1)]

    def start_dmas(c, bufset):
      r0 = row0 + c * CHUNK
      xb, yb, zb, qb, bb, sem = bufset
      return [
          pltpu.async_copy(x_hbm.at[pl.ds(r0, CHUNK)], xb, sem),
          pltpu.async_copy(y_hbm.at[pl.ds(r0, CHUNK)], yb, sem),
          pltpu.async_copy(z_hbm.at[pl.ds(r0, CHUNK)], zb, sem),
          pltpu.async_copy(q_hbm.at[pl.ds(r0, CHUNK)], qb, sem),
          pltpu.async_copy(b_hbm.at[pl.ds(r0, CHUNK)], bb, sem),
      ]

    def make_block(bufset):
      xbuf, ybuf, zbuf, qbuf, bbuf, _ = bufset

      def process_block(o0, b_new, carry):
        def fast_fn(carry):
          cur, vqx, vqy, vqz, vpx, vpy, vpz, vqs = carry
          xs, ys, zs, qs, qxs, qys, qzs = [], [], [], [], [], [], []
          for u in range(U):
            o = o0 + u * 16
            xv = xbuf[pl.ds(o, 16)]
            yv = ybuf[pl.ds(o, 16)]
            zv = zbuf[pl.ds(o, 16)]
            qv = qbuf[pl.ds(o, 16)]
            xs.append(xv); ys.append(yv); zs.append(zv); qs.append(qv)
            qxs.append(qv * xv); qys.append(qv * yv); qzs.append(qv * zv)
          return (cur,
                  vqx + _tree_sum(qxs), vqy + _tree_sum(qys),
                  vqz + _tree_sum(qzs),
                  vpx + _tree_sum(xs), vpy + _tree_sum(ys),
                  vpz + _tree_sum(zs), vqs + _tree_sum(qs))

        def slow_fn(carry):
          cur, vqx, vqy, vqz, vpx, vpy, vpz, vqs = carry
          flush(cur, vqx, vqy, vqz, vpx, vpy, vpz)
          qs = []
          for u in range(U):
            o = o0 + u * 16
            xv = xbuf[pl.ds(o, 16)]
            yv = ybuf[pl.ds(o, 16)]
            zv = zbuf[pl.ds(o, 16)]
            qv = qbuf[pl.ds(o, 16)]
            b3 = bbuf[pl.ds(o, 16)] * 3
            plsc.addupdate_scatter(acc_qp, [b3], qv * xv)
            plsc.addupdate_scatter(acc_qp, [b3 + 1], qv * yv)
            plsc.addupdate_scatter(acc_qp, [b3 + 2], qv * zv)
            plsc.addupdate_scatter(acc_p, [b3], xv)
            plsc.addupdate_scatter(acc_p, [b3 + 1], yv)
            plsc.addupdate_scatter(acc_p, [b3 + 2], zv)
            qs.append(qv)
          return (b_new, zeros, zeros, zeros, zeros, zeros, zeros,
                  vqs + _tree_sum(qs))

        return lax.cond(b_new == carry[0], fast_fn, slow_fn, carry)

      def pair_body(k, carry):
        k0 = k * (2 * BLK)
        bA = bbuf[pl.ds(k0 + BLK - 16, 16)][15]
        bB = bbuf[pl.ds(k0 + 2 * BLK - 16, 16)][15]
        carry = process_block(k0, bA, carry)
        return process_block(k0 + BLK, bB, carry)

      def tail_body(k, carry):
        k0 = k * BLK
        b_new = bbuf[pl.ds(k0 + BLK - 16, 16)][15]
        return process_block(k0, b_new, carry)

      return pair_body, tail_body

    carry = (jnp.int32(-1), zeros, zeros, zeros, zeros, zeros, zeros, zeros)
    handles = start_dmas(0, bufsets[0])
    for c in range(NCHUNK):
      for h in handles:
        h.wait()
      if c + 1 < NCHUNK:
        handles = start_dmas(c + 1, bufsets[(c + 1) % 2])
      pair_body, tail_body = make_block(bufsets[c % 2])
      carry = lax.fori_loop(0, NBLK // 2, pair_body, carry)
      carry = lax.fori_loop(NBLK - (NBLK % 2), NBLK, tail_body, carry)
    cur, vqx, vqy, vqz, vpx, vpy, vpz, vqs = carry
    flush(cur, vqx, vqy, vqz, vpx, vpy, vpz)
    qs_buf[...] = vqs

    pltpu.sync_copy(acc_qp, out_qp.at[wid])
    pltpu.sync_copy(acc_p, out_p.at[wid])
    pltpu.sync_copy(qs_buf, out_qs.at[wid])

  return body(x, y, z, q, batch)


def _combine_body(qp_ref, p_ref, qs_ref, out_ref):
  m = jnp.sum(qs_ref[...]) * (1.0 / N_POINTS)
  out_ref[...] = (jnp.sum(qp_ref[...], axis=0, keepdims=True)
                  - m * jnp.sum(p_ref[...], axis=0, keepdims=True))


def kernel(positions, q, batch):
  x = positions[:, 0]
  y = positions[:, 1]
  z = positions[:, 2]
  qp_part, p_part, qs_part = _sc_partials(x, y, z, q, batch)
  out = pl.pallas_call(
      _combine_body,
      out_shape=jax.ShapeDtypeStruct((1, ACC), jnp.float32),
  )(qp_part, p_part, qs_part)
  return out.reshape(NUM_SEG, 3)


# speedup vs baseline: 3.4786x; 1.1082x over previous
import functools

import jax
import jax.numpy as jnp
from jax import lax
from jax.experimental import pallas as pl
from jax.experimental.pallas import tpu as pltpu
from jax.experimental.pallas import tpu_sc as plsc

N_POINTS = 1600000
NUM_SEG = 1024
ACC = NUM_SEG * 3

_info = plsc.get_sparse_core_info()
NUM_CORES = _info.num_cores
NUM_SUBCORES = _info.num_subcores
NW = NUM_CORES * NUM_SUBCORES
ROWS_PER_W = N_POINTS // NW
CHUNK = 10000
VREGS = CHUNK // 16
U = 5
BLK = U * 16
NBLK = VREGS // U


def _tree_sum(vs):
  while len(vs) > 1:
    vs = [a + b for a, b in zip(vs[::2], vs[1::2])] + (
        [vs[-1]] if len(vs) % 2 else [])
  return vs[0]


def _sc_partials(x, y, z, q, batch, row_base, rows_per_w):
  mesh = plsc.VectorSubcoreMesh(core_axis_name="c", subcore_axis_name="s")
  nchunk = rows_per_w // CHUNK

  @functools.partial(
      pl.kernel,
      mesh=mesh,
      compiler_params=pltpu.CompilerParams(needs_layout_passes=False),
      out_type=[
          jax.ShapeDtypeStruct((NW, ACC), jnp.float32),
          jax.ShapeDtypeStruct((NW, ACC), jnp.float32),
          jax.ShapeDtypeStruct((NW, 16), jnp.float32),
      ],
      scratch_types=[
          pltpu.VMEM((CHUNK,), jnp.float32),
          pltpu.VMEM((CHUNK,), jnp.float32),
          pltpu.VMEM((CHUNK,), jnp.float32),
          pltpu.VMEM((CHUNK,), jnp.float32),
          pltpu.VMEM((CHUNK,), jnp.int32),
          pltpu.VMEM((CHUNK,), jnp.float32),
          pltpu.VMEM((CHUNK,), jnp.float32),
          pltpu.VMEM((CHUNK,), jnp.float32),
          pltpu.VMEM((CHUNK,), jnp.float32),
          pltpu.VMEM((CHUNK,), jnp.int32),
          pltpu.VMEM((ACC,), jnp.float32),
          pltpu.VMEM((ACC,), jnp.float32),
          pltpu.VMEM((16,), jnp.float32),
          pltpu.SemaphoreType.DMA,
          pltpu.SemaphoreType.DMA,
      ],
  )
  def body(x_hbm, y_hbm, z_hbm, q_hbm, b_hbm, out_qp, out_p, out_qs,
           xbuf0, ybuf0, zbuf0, qbuf0, bbuf0,
           xbuf1, ybuf1, zbuf1, qbuf1, bbuf1,
           acc_qp, acc_p, qs_buf, sem0, sem1):
    wid = lax.axis_index("s") * NUM_CORES + lax.axis_index("c")
    row0 = row_base + wid * rows_per_w

    zeros = jnp.zeros((16,), jnp.float32)
    zeros_i = jnp.zeros((16,), jnp.int32)

    def zero_body(j, _):
      acc_qp[pl.ds(j * 16, 16)] = zeros
      acc_p[pl.ds(j * 16, 16)] = zeros
      return 0

    lax.fori_loop(0, ACC // 16, zero_body, 0)

    def flush(cur, vqx, vqy, vqz, vpx, vpy, vpz):
      i0 = zeros_i + jnp.maximum(cur, 0) * 3
      plsc.addupdate_scatter(acc_qp, [i0], vqx)
      plsc.addupdate_scatter(acc_qp, [i0 + 1], vqy)
      plsc.addupdate_scatter(acc_qp, [i0 + 2], vqz)
      plsc.addupdate_scatter(acc_p, [i0], vpx)
      plsc.addupdate_scatter(acc_p, [i0 + 1], vpy)
      plsc.addupdate_scatter(acc_p, [i0 + 2], vpz)

    bufsets = [(xbuf0, ybuf0, zbuf0, qbuf0, bbuf0, sem0),
               (xbuf1, ybuf1, zbuf1, qbuf1, bbuf1, sem1)]

    def start_dmas(c, bufset):
      r0 = row0 + c * CHUNK
      rp = r0 - row_base
      xb, yb, zb, qb, bb, sem = bufset
      return [
          pltpu.async_copy(x_hbm.at[pl.ds(rp, CHUNK)], xb, sem),
          pltpu.async_copy(y_hbm.at[pl.ds(rp, CHUNK)], yb, sem),
          pltpu.async_copy(z_hbm.at[pl.ds(rp, CHUNK)], zb, sem),
          pltpu.async_copy(q_hbm.at[pl.ds(r0, CHUNK)], qb, sem),
          pltpu.async_copy(b_hbm.at[pl.ds(r0, CHUNK)], bb, sem),
      ]

    def make_block(bufset):
      xbuf, ybuf, zbuf, qbuf, bbuf, _ = bufset

      def process_block(o0, b_new, carry):
        def fast_fn(carry):
          cur, vqx, vqy, vqz, vpx, vpy, vpz, vqs = carry
          xs, ys, zs, qs, qxs, qys, qzs = [], [], [], [], [], [], []
          for u in range(U):
            o = o0 + u * 16
            xv = xbuf[pl.ds(o, 16)]
            yv = ybuf[pl.ds(o, 16)]
            zv = zbuf[pl.ds(o, 16)]
            qv = qbuf[pl.ds(o, 16)]
            xs.append(xv); ys.append(yv); zs.append(zv); qs.append(qv)
            qxs.append(qv * xv); qys.append(qv * yv); qzs.append(qv * zv)
          return (cur,
                  vqx + _tree_sum(qxs), vqy + _tree_sum(qys),
                  vqz + _tree_sum(qzs),
                  vpx + _tree_sum(xs), vpy + _tree_sum(ys),
                  vpz + _tree_sum(zs), vqs + _tree_sum(qs))

        def slow_fn(carry):
          cur, vqx, vqy, vqz, vpx, vpy, vpz, vqs = carry
          flush(cur, vqx, vqy, vqz, vpx, vpy, vpz)
          qs = []
          for u in range(U):
            o = o0 + u * 16
            xv = xbuf[pl.ds(o, 16)]
            yv = ybuf[pl.ds(o, 16)]
            zv = zbuf[pl.ds(o, 16)]
            qv = qbuf[pl.ds(o, 16)]
            b3 = bbuf[pl.ds(o, 16)] * 3
            plsc.addupdate_scatter(acc_qp, [b3], qv * xv)
            plsc.addupdate_scatter(acc_qp, [b3 + 1], qv * yv)
            plsc.addupdate_scatter(acc_qp, [b3 + 2], qv * zv)
            plsc.addupdate_scatter(acc_p, [b3], xv)
            plsc.addupdate_scatter(acc_p, [b3 + 1], yv)
            plsc.addupdate_scatter(acc_p, [b3 + 2], zv)
            qs.append(qv)
          return (b_new, zeros, zeros, zeros, zeros, zeros, zeros,
                  vqs + _tree_sum(qs))

        return lax.cond(b_new == carry[0], fast_fn, slow_fn, carry)

      def pair_body(k, carry):
        k0 = k * (2 * BLK)
        bA = bbuf[pl.ds(k0 + BLK - 16, 16)][15]
        bB = bbuf[pl.ds(k0 + 2 * BLK - 16, 16)][15]
        carry = process_block(k0, bA, carry)
        return process_block(k0 + BLK, bB, carry)

      def tail_body(k, carry):
        k0 = k * BLK
        b_new = bbuf[pl.ds(k0 + BLK - 16, 16)][15]
        return process_block(k0, b_new, carry)

      return pair_body, tail_body

    carry = (jnp.int32(-1), zeros, zeros, zeros, zeros, zeros, zeros, zeros)
    handles = start_dmas(0, bufsets[0])
    for c in range(nchunk):
      for h in handles:
        h.wait()
      if c + 1 < nchunk:
        handles = start_dmas(c + 1, bufsets[(c + 1) % 2])
      pair_body, tail_body = make_block(bufsets[c % 2])
      carry = lax.fori_loop(0, NBLK // 2, pair_body, carry)
      carry = lax.fori_loop(NBLK - (NBLK % 2), NBLK, tail_body, carry)
    cur, vqx, vqy, vqz, vpx, vpy, vpz, vqs = carry
    flush(cur, vqx, vqy, vqz, vpx, vpy, vpz)
    qs_buf[...] = vqs

    pltpu.sync_copy(acc_qp, out_qp.at[wid])
    pltpu.sync_copy(acc_p, out_p.at[wid])
    pltpu.sync_copy(qs_buf, out_qs.at[wid])

  return body(x, y, z, q, batch)


def _combine_body(qp_ref, p_ref, qs_ref, out_ref):
  m = jnp.sum(qs_ref[...]) * (1.0 / N_POINTS)
  out_ref[...] = (jnp.sum(qp_ref[...], axis=0, keepdims=True)
                  - m * jnp.sum(p_ref[...], axis=0, keepdims=True))


def kernel(positions, q, batch):
  n_a = 32 * 30000
  x_a = positions[:n_a, 0]
  y_a = positions[:n_a, 1]
  z_a = positions[:n_a, 2]
  qp_a, p_a, qs_a = _sc_partials(x_a, y_a, z_a, q, batch, 0, 30000)
  x_b = positions[n_a:, 0]
  y_b = positions[n_a:, 1]
  z_b = positions[n_a:, 2]
  qp_b, p_b, qs_b = _sc_partials(x_b, y_b, z_b, q, batch, n_a, 20000)
  out = pl.pallas_call(
      _combine_body,
      out_shape=jax.ShapeDtypeStruct((1, ACC), jnp.float32),
  )(jnp.concatenate([qp_a, qp_b]), jnp.concatenate([p_a, p_b]),
    jnp.concatenate([qs_a, qs_b]))
  return out.reshape(NUM_SEG, 3)


# speedup vs baseline: 3.7889x; 1.0892x over previous
import functools

import jax
import jax.numpy as jnp
from jax import lax
from jax.experimental import pallas as pl
from jax.experimental.pallas import tpu as pltpu
from jax.experimental.pallas import tpu_sc as plsc

N_POINTS = 1600000
NUM_SEG = 1024
ACC = NUM_SEG * 3

_info = plsc.get_sparse_core_info()
NUM_CORES = _info.num_cores
NUM_SUBCORES = _info.num_subcores
NW = NUM_CORES * NUM_SUBCORES
ROWS_PER_W = N_POINTS // NW
CHUNK = 10000
VREGS = CHUNK // 16
U = 5
BLK = U * 16
NBLK = VREGS // U


def _tree_sum(vs):
  while len(vs) > 1:
    vs = [a + b for a, b in zip(vs[::2], vs[1::2])] + (
        [vs[-1]] if len(vs) % 2 else [])
  return vs[0]


def _sc_partials(x, y, z, q, batch, row_base, rows_per_w):
  mesh = plsc.VectorSubcoreMesh(core_axis_name="c", subcore_axis_name="s")
  nchunk = rows_per_w // CHUNK

  @functools.partial(
      pl.kernel,
      mesh=mesh,
      compiler_params=pltpu.CompilerParams(needs_layout_passes=False),
      out_type=[
          jax.ShapeDtypeStruct((NW, ACC), jnp.float32),
          jax.ShapeDtypeStruct((NW, ACC), jnp.float32),
          jax.ShapeDtypeStruct((NW, 16), jnp.float32),
      ],
      scratch_types=[
          pltpu.VMEM((CHUNK,), jnp.float32),
          pltpu.VMEM((CHUNK,), jnp.float32),
          pltpu.VMEM((CHUNK,), jnp.float32),
          pltpu.VMEM((CHUNK,), jnp.float32),
          pltpu.VMEM((CHUNK,), jnp.int32),
          pltpu.VMEM((CHUNK,), jnp.float32),
          pltpu.VMEM((CHUNK,), jnp.float32),
          pltpu.VMEM((CHUNK,), jnp.float32),
          pltpu.VMEM((CHUNK,), jnp.float32),
          pltpu.VMEM((CHUNK,), jnp.int32),
          pltpu.VMEM((ACC,), jnp.float32),
          pltpu.VMEM((ACC,), jnp.float32),
          pltpu.VMEM((16,), jnp.float32),
          pltpu.SemaphoreType.DMA,
          pltpu.SemaphoreType.DMA,
      ],
  )
  def body(x_hbm, y_hbm, z_hbm, q_hbm, b_hbm, out_qp, out_p, out_qs,
           xbuf0, ybuf0, zbuf0, qbuf0, bbuf0,
           xbuf1, ybuf1, zbuf1, qbuf1, bbuf1,
           acc_qp, acc_p, qs_buf, sem0, sem1):
    wid = lax.axis_index("s") * NUM_CORES + lax.axis_index("c")
    row0 = row_base + wid * rows_per_w

    zeros = jnp.zeros((16,), jnp.float32)
    zeros_i = jnp.zeros((16,), jnp.int32)

    def zero_body(j, _):
      acc_qp[pl.ds(j * 16, 16)] = zeros
      acc_p[pl.ds(j * 16, 16)] = zeros
      return 0

    lax.fori_loop(0, ACC // 16, zero_body, 0)

    def flush(cur, vqx, vqy, vqz, vpx, vpy, vpz):
      i0 = zeros_i + jnp.maximum(cur, 0) * 3
      plsc.addupdate_scatter(acc_qp, [i0], vqx)
      plsc.addupdate_scatter(acc_qp, [i0 + 1], vqy)
      plsc.addupdate_scatter(acc_qp, [i0 + 2], vqz)
      plsc.addupdate_scatter(acc_p, [i0], vpx)
      plsc.addupdate_scatter(acc_p, [i0 + 1], vpy)
      plsc.addupdate_scatter(acc_p, [i0 + 2], vpz)

    bufsets = [(xbuf0, ybuf0, zbuf0, qbuf0, bbuf0, sem0),
               (xbuf1, ybuf1, zbuf1, qbuf1, bbuf1, sem1)]

    def start_dmas(c, bufset):
      r0 = row0 + c * CHUNK
      rp = r0 - row_base
      xb, yb, zb, qb, bb, sem = bufset
      return [
          pltpu.async_copy(x_hbm.at[pl.ds(rp, CHUNK)], xb, sem),
          pltpu.async_copy(y_hbm.at[pl.ds(rp, CHUNK)], yb, sem),
          pltpu.async_copy(z_hbm.at[pl.ds(rp, CHUNK)], zb, sem),
          pltpu.async_copy(q_hbm.at[pl.ds(r0, CHUNK)], qb, sem),
          pltpu.async_copy(b_hbm.at[pl.ds(r0, CHUNK)], bb, sem),
      ]

    def make_block(bufset):
      xbuf, ybuf, zbuf, qbuf, bbuf, _ = bufset

      def process_block(o0, b_new, carry):
        def fast_fn(carry):
          cur, vqx, vqy, vqz, vpx, vpy, vpz, vqs = carry
          xs, ys, zs, qs, qxs, qys, qzs = [], [], [], [], [], [], []
          for u in range(U):
            o = o0 + u * 16
            xv = xbuf[pl.ds(o, 16)]
            yv = ybuf[pl.ds(o, 16)]
            zv = zbuf[pl.ds(o, 16)]
            qv = qbuf[pl.ds(o, 16)]
            xs.append(xv); ys.append(yv); zs.append(zv); qs.append(qv)
            qxs.append(qv * xv); qys.append(qv * yv); qzs.append(qv * zv)
          return (cur,
                  vqx + _tree_sum(qxs), vqy + _tree_sum(qys),
                  vqz + _tree_sum(qzs),
                  vpx + _tree_sum(xs), vpy + _tree_sum(ys),
                  vpz + _tree_sum(zs), vqs + _tree_sum(qs))

        def slow_fn(carry):
          cur, vqx, vqy, vqz, vpx, vpy, vpz, vqs = carry
          flush(cur, vqx, vqy, vqz, vpx, vpy, vpz)
          qs = []
          for u in range(U):
            o = o0 + u * 16
            xv = xbuf[pl.ds(o, 16)]
            yv = ybuf[pl.ds(o, 16)]
            zv = zbuf[pl.ds(o, 16)]
            qv = qbuf[pl.ds(o, 16)]
            b3 = bbuf[pl.ds(o, 16)] * 3
            plsc.addupdate_scatter(acc_qp, [b3], qv * xv)
            plsc.addupdate_scatter(acc_qp, [b3 + 1], qv * yv)
            plsc.addupdate_scatter(acc_qp, [b3 + 2], qv * zv)
            plsc.addupdate_scatter(acc_p, [b3], xv)
            plsc.addupdate_scatter(acc_p, [b3 + 1], yv)
            plsc.addupdate_scatter(acc_p, [b3 + 2], zv)
            qs.append(qv)
          return (b_new, zeros, zeros, zeros, zeros, zeros, zeros,
                  vqs + _tree_sum(qs))

        return lax.cond(b_new == carry[0], fast_fn, slow_fn, carry)

      def pair_body(k, carry):
        k0 = k * (2 * BLK)
        bA = bbuf[pl.ds(k0 + BLK - 16, 16)][15]
        bB = bbuf[pl.ds(k0 + 2 * BLK - 16, 16)][15]
        carry = process_block(k0, bA, carry)
        return process_block(k0 + BLK, bB, carry)

      def tail_body(k, carry):
        k0 = k * BLK
        b_new = bbuf[pl.ds(k0 + BLK - 16, 16)][15]
        return process_block(k0, b_new, carry)

      return pair_body, tail_body

    carry = (jnp.int32(-1), zeros, zeros, zeros, zeros, zeros, zeros, zeros)
    handles = start_dmas(0, bufsets[0])
    for c in range(nchunk):
      for h in handles:
        h.wait()
      if c + 1 < nchunk:
        handles = start_dmas(c + 1, bufsets[(c + 1) % 2])
      pair_body, tail_body = make_block(bufsets[c % 2])
      carry = lax.fori_loop(0, NBLK // 2, pair_body, carry)
      carry = lax.fori_loop(NBLK - (NBLK % 2), NBLK, tail_body, carry)
    cur, vqx, vqy, vqz, vpx, vpy, vpz, vqs = carry
    flush(cur, vqx, vqy, vqz, vpx, vpy, vpz)
    qs_buf[...] = vqs

    pltpu.sync_copy(acc_qp, out_qp.at[wid])
    pltpu.sync_copy(acc_p, out_p.at[wid])
    pltpu.sync_copy(qs_buf, out_qs.at[wid])

  return body(x, y, z, q, batch)


def _combine_body(qp_ref, p_ref, qs_ref, out_ref):
  m = jnp.sum(qs_ref[...]) * (1.0 / N_POINTS)
  out_ref[...] = (jnp.sum(qp_ref[...], axis=0, keepdims=True)
                  - m * jnp.sum(p_ref[...], axis=0, keepdims=True))


def kernel(positions, q, batch):
  qps, ps, qss = [], [], []
  base = 0
  for rows_w in (20000, 20000, 10000):
    n_part = NW * rows_w
    xs = positions[base:base + n_part, 0]
    ys = positions[base:base + n_part, 1]
    zs = positions[base:base + n_part, 2]
    qp_i, p_i, qs_i = _sc_partials(xs, ys, zs, q, batch, base, rows_w)
    qps.append(qp_i); ps.append(p_i); qss.append(qs_i)
    base += n_part
  out = pl.pallas_call(
      _combine_body,
      out_shape=jax.ShapeDtypeStruct((1, ACC), jnp.float32),
  )(jnp.concatenate(qps), jnp.concatenate(ps), jnp.concatenate(qss))
  return out.reshape(NUM_SEG, 3)
